# baseline (speedup 1.0000x reference)
import functools

import jax
import jax.numpy as jnp
from jax import lax
from jax.experimental import pallas as pl
from jax.experimental.pallas import tpu as pltpu
from jax.experimental.pallas import tpu_sc as plsc

B = 16384
EMB = 128
NC = 2
NS = 16
NW = NC * NS
BPW = B // NW
CH = BPW // 128


def _sc_gather(uid2, iid2, user_emb, item_emb):
    mesh = plsc.VectorSubcoreMesh(core_axis_name="c", subcore_axis_name="s")

    @functools.partial(
        pl.kernel,
        mesh=mesh,
        out_type=jax.ShapeDtypeStruct((B, 2 * EMB), jnp.float32),
        scratch_types=[
            pltpu.VMEM((CH, 128), jnp.int32),
            pltpu.VMEM((CH, 128), jnp.int32),
            pltpu.VMEM((BPW, EMB), jnp.float32),
            pltpu.SemaphoreType.DMA,
        ],
    )
    def k(uid_hbm, iid_hbm, uemb_hbm, iemb_hbm, x_out,
          uidx_v, iidx_v, rows_v, sem):
        wid = lax.axis_index("s") * NC + lax.axis_index("c")
        base = wid * BPW
        pltpu.sync_copy(uid_hbm.at[pl.ds(wid * CH, CH)], uidx_v)
        pltpu.sync_copy(iid_hbm.at[pl.ds(wid * CH, CH)], iidx_v)
        cps = [
            pltpu.async_copy(uemb_hbm.at[uidx_v.at[j]],
                             rows_v.at[pl.ds(j * 128, 128)], sem)
            for j in range(CH)
        ]
        for c in cps:
            c.wait()
        pltpu.sync_copy(rows_v, x_out.at[pl.ds(base, BPW), pl.ds(0, EMB)])
        cps = [
            pltpu.async_copy(iemb_hbm.at[iidx_v.at[j]],
                             rows_v.at[pl.ds(j * 128, 128)], sem)
            for j in range(CH)
        ]
        for c in cps:
            c.wait()
        pltpu.sync_copy(rows_v, x_out.at[pl.ds(base, BPW), pl.ds(EMB, EMB)])

    return k(uid2, iid2, user_emb, item_emb)


BLK = 2048


def _mlp_body(x_ref, w1_ref, b1_ref, w2_ref, b2_ref,
              w3_ref, b3_ref, wo_ref, bo_ref, out_ref):
    bf = jnp.bfloat16
    h = jnp.maximum(
        jnp.dot(x_ref[...].astype(bf), w1_ref[...].astype(bf),
                preferred_element_type=jnp.float32) + b1_ref[...], 0.0)
    h = jnp.maximum(
        jnp.dot(h.astype(bf), w2_ref[...].astype(bf),
                preferred_element_type=jnp.float32)
        + b2_ref[...], 0.0)
    h = jnp.maximum(
        jnp.dot(h.astype(bf), w3_ref[...].astype(bf),
                preferred_element_type=jnp.float32)
        + b3_ref[...], 0.0)
    out_ref[...] = jnp.sum(h * wo_ref[...], axis=1) + bo_ref[0, 0]


def _tc_mlp(x, W1, b1, W2, b2, W3, b3, wo_row, bo):
    full = lambda shape: pl.BlockSpec(shape, lambda i: (0, 0))
    return pl.pallas_call(
        _mlp_body,
        grid=(B // BLK,),
        in_specs=[
            pl.BlockSpec((BLK, 2 * EMB), lambda i: (i, 0)),
            full((256, 256)), full((1, 256)),
            full((256, 128)), full((1, 128)),
            full((128, 64)), full((1, 64)),
            full((1, 64)), full((1, 1)),
        ],
        out_specs=pl.BlockSpec((BLK,), lambda i: (i,)),
        out_shape=jax.ShapeDtypeStruct((B,), jnp.float32),
    )(x, W1, b1, W2, b2, W3, b3, wo_row, bo)


def kernel(user_id, item_id, user_emb, item_emb, W1, b1, W2, b2, W3, b3,
           Wo, bo):
    uid2 = user_id.astype(jnp.int32).reshape(NW * CH, 128)
    iid2 = item_id.astype(jnp.int32).reshape(NW * CH, 128)
    x = _sc_gather(uid2, iid2, user_emb, item_emb)
    return _tc_mlp(x, W1, b1.reshape(1, 256),
                   W2, b2.reshape(1, 128), W3, b3.reshape(1, 64),
                   Wo.reshape(1, 64), bo.reshape(1, 1))


# speedup vs baseline: 1.0139x; 1.0139x over previous
import functools

import jax
import jax.numpy as jnp
from jax import lax
from jax.experimental import pallas as pl
from jax.experimental.pallas import tpu as pltpu
from jax.experimental.pallas import tpu_sc as plsc

B = 16384
EMB = 128
NC = 2
NS = 16
NW = NC * NS
CHUNKS = 2
BC = B // CHUNKS
BPW = BC // NW
CH = BPW // 128


def _sc_gather(uid2, iid2, user_emb, item_emb):
    mesh = plsc.VectorSubcoreMesh(core_axis_name="c", subcore_axis_name="s")

    @functools.partial(
        pl.kernel,
        mesh=mesh,
        out_type=jax.ShapeDtypeStruct((BC, 2 * EMB), jnp.float32),
        scratch_types=[
            pltpu.VMEM((CH, 128), jnp.int32),
            pltpu.VMEM((CH, 128), jnp.int32),
            pltpu.VMEM((BPW, EMB), jnp.float32),
            pltpu.SemaphoreType.DMA,
        ],
    )
    def k(uid_hbm, iid_hbm, uemb_hbm, iemb_hbm, x_out,
          uidx_v, iidx_v, rows_v, sem):
        wid = lax.axis_index("s") * NC + lax.axis_index("c")
        base = wid * BPW
        pltpu.sync_copy(uid_hbm.at[pl.ds(wid * CH, CH)], uidx_v)
        pltpu.sync_copy(iid_hbm.at[pl.ds(wid * CH, CH)], iidx_v)
        cps = [
            pltpu.async_copy(uemb_hbm.at[uidx_v.at[j]],
                             rows_v.at[pl.ds(j * 128, 128)], sem)
            for j in range(CH)
        ]
        for c in cps:
            c.wait()
        pltpu.sync_copy(rows_v, x_out.at[pl.ds(base, BPW), pl.ds(0, EMB)])
        cps = [
            pltpu.async_copy(iemb_hbm.at[iidx_v.at[j]],
                             rows_v.at[pl.ds(j * 128, 128)], sem)
            for j in range(CH)
        ]
        for c in cps:
            c.wait()
        pltpu.sync_copy(rows_v, x_out.at[pl.ds(base, BPW), pl.ds(EMB, EMB)])

    return k(uid2, iid2, user_emb, item_emb)


BLK = 2048


def _mlp_body(x_ref, w1_ref, b1_ref, w2_ref, b2_ref,
              w3_ref, b3_ref, wo_ref, bo_ref, out_ref):
    bf = jnp.bfloat16
    h = jnp.maximum(
        jnp.dot(x_ref[...].astype(bf), w1_ref[...].astype(bf),
                preferred_element_type=jnp.float32) + b1_ref[...], 0.0)
    h = jnp.maximum(
        jnp.dot(h.astype(bf), w2_ref[...].astype(bf),
                preferred_element_type=jnp.float32)
        + b2_ref[...], 0.0)
    h = jnp.maximum(
        jnp.dot(h.astype(bf), w3_ref[...].astype(bf),
                preferred_element_type=jnp.float32)
        + b3_ref[...], 0.0)
    out_ref[...] = jnp.sum(h * wo_ref[...], axis=1) + bo_ref[0, 0]


def _tc_mlp(x, W1, b1, W2, b2, W3, b3, wo_row, bo):
    full = lambda shape: pl.BlockSpec(shape, lambda i: (0, 0))
    return pl.pallas_call(
        _mlp_body,
        grid=(BC // BLK,),
        in_specs=[
            pl.BlockSpec((BLK, 2 * EMB), lambda i: (i, 0)),
            full((256, 256)), full((1, 256)),
            full((256, 128)), full((1, 128)),
            full((128, 64)), full((1, 64)),
            full((1, 64)), full((1, 1)),
        ],
        out_specs=pl.BlockSpec((BLK,), lambda i: (i,)),
        out_shape=jax.ShapeDtypeStruct((BC,), jnp.float32),
    )(x, W1, b1, W2, b2, W3, b3, wo_row, bo)


def kernel(user_id, item_id, user_emb, item_emb, W1, b1, W2, b2, W3, b3,
           Wo, bo):
    uid3 = user_id.astype(jnp.int32).reshape(CHUNKS, NW * CH, 128)
    iid3 = item_id.astype(jnp.int32).reshape(CHUNKS, NW * CH, 128)
    b1r, b2r, b3r = b1.reshape(1, 256), b2.reshape(1, 128), b3.reshape(1, 64)
    wo_row, bor = Wo.reshape(1, 64), bo.reshape(1, 1)
    outs = []
    for c in range(CHUNKS):
        x = _sc_gather(uid3[c], iid3[c], user_emb, item_emb)
        outs.append(_tc_mlp(x, W1, b1r, W2, b2r, W3, b3r, wo_row, bor))
    return jnp.concatenate(outs)
